# Initial kernel scaffold; baseline (speedup 1.0000x reference)
import jax
import jax.numpy as jnp
from jax.experimental import pallas as pl


def kernel(inp, w_gate, b_gate, W1, b1, W2, b2):
    raise NotImplementedError("write your pallas kernel here")


# speedup vs baseline: 1.4578x; 1.4578x over previous
import functools

import jax
import jax.numpy as jnp
from jax import lax
from jax.experimental import pallas as pl
from jax.experimental.pallas import tpu as pltpu
from jax.experimental.pallas import tpu_sc as plsc

E = 8
K = 2
D = 768
F = 3072
N = 2048
S = N * K
B = 256
NB = S // B + E
PTOT = NB * B
FB = 768
NF = F // FB

NC, NS = 2, 16
NW = NC * NS


def _gate_body(x_ref, wg_ref, bg_ref, gs_ref, ti_ref):
    logits = jnp.dot(x_ref[...], wg_ref[...],
                     preferred_element_type=jnp.float32) + bg_ref[...]
    col = lax.broadcasted_iota(jnp.int32, (N, E), 1)
    v0 = jnp.max(logits, axis=1, keepdims=True)
    i0 = jnp.min(jnp.where(logits == v0, col, E), axis=1, keepdims=True)
    masked = jnp.where(col == i0, -jnp.inf, logits)
    v1 = jnp.max(masked, axis=1, keepdims=True)
    i1 = jnp.min(jnp.where(masked == v1, col, E), axis=1, keepdims=True)
    e = jnp.exp(v1 - v0)
    s0 = 1.0 / (1.0 + e)
    gs_ref[...] = jnp.concatenate([s0, 1.0 - s0], axis=1)
    ti_ref[...] = jnp.concatenate([i0, i1], axis=1)


def _gate(inp, w_gate, b_gate):
    return pl.pallas_call(
        _gate_body,
        out_shape=(jax.ShapeDtypeStruct((N, K), jnp.float32),
                   jax.ShapeDtypeStruct((N, K), jnp.int32)),
    )(inp, w_gate, b_gate.reshape(1, E))


_CH = 64


@functools.cache
def _make_dispatch():
    @functools.partial(
        pl.kernel,
        out_type=jax.ShapeDtypeStruct((PTOT, D), jnp.float32),
        mesh=plsc.VectorSubcoreMesh(core_axis_name="c", subcore_axis_name="s",
                                    num_cores=NC, num_subcores=NS),
        scratch_types=[
            pltpu.VMEM((_CH,), jnp.int32),
            pltpu.VMEM((_CH, D), jnp.float32),
            pltpu.SemaphoreType.DMA,
        ],
    )
    def _dispatch(inp_h, tok_h, x_h, idx_v, rows_v, sem):
        wid = lax.axis_index("s") * NC + lax.axis_index("c")
        per = PTOT // NW
        base = wid * per
        for c in range(per // _CH):
            pltpu.sync_copy(tok_h.at[pl.ds(base + c * _CH, _CH)], idx_v)
            pltpu.async_copy(inp_h.at[idx_v], rows_v, sem).wait()
            pltpu.sync_copy(rows_v, x_h.at[pl.ds(base + c * _CH, _CH)])

    return _dispatch


def _ffn_body(be_ref, ba_ref, x_ref, s_ref, w1_ref, b1_ref, w2_ref, b2_ref,
              y_ref):
    g = pl.program_id(0)
    f = pl.program_id(1)

    @pl.when(ba_ref[g] == 1)
    def _():
        h = jnp.dot(x_ref[...], w1_ref[0],
                    preferred_element_type=jnp.float32) + b1_ref[0, 0, :]
        part = jnp.dot(jax.nn.gelu(h), w2_ref[0],
                       preferred_element_type=jnp.float32)

        @pl.when(f == 0)
        def _():
            y_ref[...] = (part + b2_ref[0, 0, :]) * s_ref[...]

        @pl.when(f > 0)
        def _():
            y_ref[...] += part * s_ref[...]


def _ffn(x_sorted, score_col, W1, b1, W2, b2, block_e, block_a):
    grid_spec = pltpu.PrefetchScalarGridSpec(
        num_scalar_prefetch=2,
        grid=(NB, NF),
        in_specs=[
            pl.BlockSpec((B, D), lambda g, f, be, ba: (g, 0)),
            pl.BlockSpec((B, 1), lambda g, f, be, ba: (g, 0)),
            pl.BlockSpec((1, D, FB), lambda g, f, be, ba: (be[g], 0, f)),
            pl.BlockSpec((1, 1, FB), lambda g, f, be, ba: (be[g], 0, f)),
            pl.BlockSpec((1, FB, D), lambda g, f, be, ba: (be[g], f, 0)),
            pl.BlockSpec((1, 1, D), lambda g, f, be, ba: (be[g], 0, 0)),
        ],
        out_specs=pl.BlockSpec((B, D), lambda g, f, be, ba: (g, 0)),
    )
    return pl.pallas_call(
        _ffn_body,
        grid_spec=grid_spec,
        out_shape=jax.ShapeDtypeStruct((PTOT, D), jnp.float32),
        compiler_params=pltpu.CompilerParams(
            dimension_semantics=("arbitrary", "arbitrary")),
    )(block_e, block_a, x_sorted, score_col, W1, b1.reshape(E, 1, F), W2,
      b2.reshape(E, 1, D))


_TCH = 16


@functools.cache
def _make_combine():
    @functools.partial(
        pl.kernel,
        out_type=jax.ShapeDtypeStruct((N, D), jnp.float32),
        mesh=plsc.VectorSubcoreMesh(core_axis_name="c", subcore_axis_name="s",
                                    num_cores=NC, num_subcores=NS),
        scratch_types=[
            pltpu.VMEM((_TCH,), jnp.int32),
            pltpu.VMEM((_TCH,), jnp.int32),
            pltpu.VMEM((_TCH, D), jnp.float32),
            pltpu.VMEM((_TCH, D), jnp.float32),
            pltpu.VMEM((_TCH, D), jnp.float32),
            pltpu.SemaphoreType.DMA,
        ],
    )
    def _combine(y_h, pe_h, po_h, out_h, pe_v, po_v, a_v, b_v, o_v, sem):
        wid = lax.axis_index("s") * NC + lax.axis_index("c")
        per = N // NW
        for c in range(per // _TCH):
            tb = wid * per + c * _TCH
            pltpu.sync_copy(pe_h.at[pl.ds(tb, _TCH)], pe_v)
            pltpu.sync_copy(po_h.at[pl.ds(tb, _TCH)], po_v)
            pltpu.async_copy(y_h.at[pe_v], a_v, sem).wait()
            pltpu.async_copy(y_h.at[po_v], b_v, sem).wait()

            def _add_row(t, _):
                for cc in range(D // 16):
                    sl = pl.ds(cc * 16, 16)
                    o_v[t, sl] = a_v[t, sl] + b_v[t, sl]
                return 0

            lax.fori_loop(0, _TCH, _add_row, 0)
            pltpu.sync_copy(o_v, out_h.at[pl.ds(tb, _TCH)])

    return _combine


def kernel(inp, w_gate, b_gate, W1, b1, W2, b2):
    gate_score, top_idx = _gate(inp, w_gate, b_gate)

    flat_idx = top_idx.reshape(-1)
    sort_idx = jnp.argsort(flat_idx, stable=True)
    sorted_e = flat_idx[sort_idx]
    counts = jnp.sum(flat_idx[:, None] == jnp.arange(E)[None, :], axis=0)
    blocks_per_e = (counts + B - 1) // B
    padded_counts = blocks_per_e * B
    p_off = jnp.concatenate(
        [jnp.zeros((1,), jnp.int32),
         jnp.cumsum(padded_counts)[:-1].astype(jnp.int32)])
    off = jnp.concatenate(
        [jnp.zeros((1,), jnp.int32),
         jnp.cumsum(counts)[:-1].astype(jnp.int32)])
    r = jnp.arange(S, dtype=jnp.int32)
    pos_sorted = p_off[sorted_e] + (r - off[sorted_e])
    tok_of_pos = jnp.zeros((PTOT,), jnp.int32).at[pos_sorted].set(
        (sort_idx // K).astype(jnp.int32))
    pos_of_slot = jnp.zeros((S,), jnp.int32).at[sort_idx].set(pos_sorted)
    score_of_pos = jnp.zeros((PTOT,), jnp.float32).at[pos_sorted].set(
        gate_score.reshape(-1)[sort_idx])

    num_active = jnp.sum(blocks_per_e).astype(jnp.int32)
    be_raw = jnp.minimum(
        jnp.searchsorted(jnp.cumsum(blocks_per_e), jnp.arange(NB),
                         side="right"), E - 1).astype(jnp.int32)
    last_e = jnp.max(jnp.where(counts > 0, jnp.arange(E), 0)).astype(jnp.int32)
    block_e = jnp.where(jnp.arange(NB) < num_active, be_raw, last_e)
    block_a = (jnp.arange(NB) < num_active).astype(jnp.int32)

    x_sorted = _make_dispatch()(inp, tok_of_pos)
    y_sorted = _ffn(x_sorted, score_of_pos[:, None], W1, b1, W2, b2,
                    block_e, block_a)
    out = _make_combine()(y_sorted, pos_of_slot[0::2], pos_of_slot[1::2])
    return out


# speedup vs baseline: 1.6633x; 1.1410x over previous
import functools

import jax
import jax.numpy as jnp
from jax import lax
from jax.experimental import pallas as pl
from jax.experimental.pallas import tpu as pltpu
from jax.experimental.pallas import tpu_sc as plsc

E = 8
K = 2
D = 768
F = 3072
N = 2048
S = N * K
B = 256
NB = S // B + E
PTOT = NB * B
FB = 768
NF = F // FB

NC, NS = 2, 16
NW = NC * NS


def _gate_body(x_ref, wg_ref, bg_ref, gs_ref, ti_ref):
    logits = jnp.dot(x_ref[...], wg_ref[...],
                     preferred_element_type=jnp.float32) + bg_ref[...]
    col = lax.broadcasted_iota(jnp.int32, (N, E), 1)
    v0 = jnp.max(logits, axis=1, keepdims=True)
    i0 = jnp.min(jnp.where(logits == v0, col, E), axis=1, keepdims=True)
    masked = jnp.where(col == i0, -jnp.inf, logits)
    v1 = jnp.max(masked, axis=1, keepdims=True)
    i1 = jnp.min(jnp.where(masked == v1, col, E), axis=1, keepdims=True)
    e = jnp.exp(v1 - v0)
    s0 = 1.0 / (1.0 + e)
    gs_ref[...] = jnp.concatenate([s0, 1.0 - s0], axis=1)
    ti_ref[...] = jnp.concatenate([i0, i1], axis=1)


def _gate(inp, w_gate, b_gate):
    return pl.pallas_call(
        _gate_body,
        out_shape=(jax.ShapeDtypeStruct((N, K), jnp.float32),
                   jax.ShapeDtypeStruct((N, K), jnp.int32)),
    )(inp, w_gate, b_gate.reshape(1, E))


_CH = 64


@functools.cache
def _make_gather(n_out):
    @functools.partial(
        pl.kernel,
        out_type=jax.ShapeDtypeStruct((n_out, D), jnp.float32),
        mesh=plsc.VectorSubcoreMesh(core_axis_name="c", subcore_axis_name="s",
                                    num_cores=NC, num_subcores=NS),
        scratch_types=[
            pltpu.VMEM((_CH,), jnp.int32),
            pltpu.VMEM((_CH, D), jnp.float32),
            pltpu.SemaphoreType.DMA,
        ],
    )
    def _gather(src_h, idx_h, out_h, idx_v, rows_v, sem):
        wid = lax.axis_index("s") * NC + lax.axis_index("c")
        per = n_out // NW
        base = wid * per
        for c in range(per // _CH):
            pltpu.sync_copy(idx_h.at[pl.ds(base + c * _CH, _CH)], idx_v)
            pltpu.async_copy(src_h.at[idx_v], rows_v, sem).wait()
            pltpu.sync_copy(rows_v, out_h.at[pl.ds(base + c * _CH, _CH)])

    return _gather


def _ffn_body(be_ref, ba_ref, x_ref, s_ref, w1_ref, b1_ref, w2_ref, b2_ref,
              y_ref):
    g = pl.program_id(0)

    @pl.when(ba_ref[g] == 1)
    def _():
        h = jnp.dot(x_ref[...], w1_ref[0],
                    preferred_element_type=jnp.float32) + b1_ref[0, 0, :]
        y = jnp.dot(jax.nn.gelu(h), w2_ref[0],
                    preferred_element_type=jnp.float32)
        y_ref[...] = (y + b2_ref[0, 0, :]) * s_ref[...]


def _ffn(x_sorted, score_col, W1, b1, W2, b2, block_e, block_a):
    grid_spec = pltpu.PrefetchScalarGridSpec(
        num_scalar_prefetch=2,
        grid=(NB,),
        in_specs=[
            pl.BlockSpec((B, D), lambda g, be, ba: (g, 0)),
            pl.BlockSpec((B, 1), lambda g, be, ba: (g, 0)),
            pl.BlockSpec((1, D, F), lambda g, be, ba: (be[g], 0, 0)),
            pl.BlockSpec((1, 1, F), lambda g, be, ba: (be[g], 0, 0)),
            pl.BlockSpec((1, F, D), lambda g, be, ba: (be[g], 0, 0)),
            pl.BlockSpec((1, 1, D), lambda g, be, ba: (be[g], 0, 0)),
        ],
        out_specs=pl.BlockSpec((B, D), lambda g, be, ba: (g, 0)),
    )
    return pl.pallas_call(
        _ffn_body,
        grid_spec=grid_spec,
        out_shape=jax.ShapeDtypeStruct((PTOT, D), jnp.float32),
        compiler_params=pltpu.CompilerParams(
            dimension_semantics=("arbitrary",)),
    )(block_e, block_a, x_sorted, score_col, W1, b1.reshape(E, 1, F), W2,
      b2.reshape(E, 1, D))


_BT = 512


def _pair_add_body(g_ref, o_ref):
    o_ref[...] = g_ref[:, 0, :] + g_ref[:, 1, :]


def _pair_add(g):
    return pl.pallas_call(
        _pair_add_body,
        grid=(N // _BT,),
        in_specs=[pl.BlockSpec((_BT, K, D), lambda i: (i, 0, 0))],
        out_specs=pl.BlockSpec((_BT, D), lambda i: (i, 0)),
        out_shape=jax.ShapeDtypeStruct((N, D), jnp.float32),
    )(g)


def kernel(inp, w_gate, b_gate, W1, b1, W2, b2):
    gate_score, top_idx = _gate(inp, w_gate, b_gate)

    flat_idx = top_idx.reshape(-1)
    sort_idx = jnp.argsort(flat_idx, stable=True)
    sorted_e = flat_idx[sort_idx]
    counts = jnp.sum(flat_idx[:, None] == jnp.arange(E)[None, :], axis=0)
    blocks_per_e = (counts + B - 1) // B
    padded_counts = blocks_per_e * B
    p_off = jnp.concatenate(
        [jnp.zeros((1,), jnp.int32),
         jnp.cumsum(padded_counts)[:-1].astype(jnp.int32)])
    off = jnp.concatenate(
        [jnp.zeros((1,), jnp.int32),
         jnp.cumsum(counts)[:-1].astype(jnp.int32)])
    r = jnp.arange(S, dtype=jnp.int32)
    pos_sorted = p_off[sorted_e] + (r - off[sorted_e])
    tok_of_pos = jnp.zeros((PTOT,), jnp.int32).at[pos_sorted].set(
        (sort_idx // K).astype(jnp.int32))
    pos_of_slot = jnp.zeros((S,), jnp.int32).at[sort_idx].set(pos_sorted)
    score_of_pos = jnp.zeros((PTOT,), jnp.float32).at[pos_sorted].set(
        gate_score.reshape(-1)[sort_idx])

    num_active = jnp.sum(blocks_per_e).astype(jnp.int32)
    be_raw = jnp.minimum(
        jnp.searchsorted(jnp.cumsum(blocks_per_e), jnp.arange(NB),
                         side="right"), E - 1).astype(jnp.int32)
    last_e = jnp.max(jnp.where(counts > 0, jnp.arange(E), 0)).astype(jnp.int32)
    block_e = jnp.where(jnp.arange(NB) < num_active, be_raw, last_e)
    block_a = (jnp.arange(NB) < num_active).astype(jnp.int32)

    x_sorted = _make_gather(PTOT)(inp, tok_of_pos)
    y_sorted = _ffn(x_sorted, score_of_pos[:, None], W1, b1, W2, b2,
                    block_e, block_a)
    g = _make_gather(S)(y_sorted, pos_of_slot)
    return _pair_add(g.reshape(N, K, D))


# speedup vs baseline: 1.6897x; 1.0159x over previous
import functools

import jax
import jax.numpy as jnp
from jax import lax
from jax.experimental import pallas as pl
from jax.experimental.pallas import tpu as pltpu
from jax.experimental.pallas import tpu_sc as plsc

E = 8
K = 2
D = 768
F = 3072
N = 2048
S = N * K
B = 256
NB = S // B + E
PTOT = NB * B
FB = 768
NF = F // FB

NC, NS = 2, 16
NW = NC * NS


def _gate_body(x_ref, wg_ref, bg_ref, gs_ref, ti_ref):
    logits = jnp.dot(x_ref[...], wg_ref[...],
                     preferred_element_type=jnp.float32) + bg_ref[...]
    col = lax.broadcasted_iota(jnp.int32, (N, E), 1)
    v0 = jnp.max(logits, axis=1, keepdims=True)
    i0 = jnp.min(jnp.where(logits == v0, col, E), axis=1, keepdims=True)
    masked = jnp.where(col == i0, -jnp.inf, logits)
    v1 = jnp.max(masked, axis=1, keepdims=True)
    i1 = jnp.min(jnp.where(masked == v1, col, E), axis=1, keepdims=True)
    e = jnp.exp(v1 - v0)
    s0 = 1.0 / (1.0 + e)
    gs_ref[...] = jnp.concatenate([s0, 1.0 - s0], axis=1)
    ti_ref[...] = jnp.concatenate([i0, i1], axis=1)


def _gate(inp, w_gate, b_gate):
    return pl.pallas_call(
        _gate_body,
        out_shape=(jax.ShapeDtypeStruct((N, K), jnp.float32),
                   jax.ShapeDtypeStruct((N, K), jnp.int32)),
    )(inp, w_gate, b_gate.reshape(1, E))


_CH = 64


@functools.cache
def _make_gather(n_out):
    @functools.partial(
        pl.kernel,
        out_type=jax.ShapeDtypeStruct((n_out, D), jnp.float32),
        mesh=plsc.VectorSubcoreMesh(core_axis_name="c", subcore_axis_name="s",
                                    num_cores=NC, num_subcores=NS),
        scratch_types=[
            pltpu.VMEM((_CH,), jnp.int32),
            pltpu.VMEM((_CH, D), jnp.float32),
            pltpu.SemaphoreType.DMA,
        ],
    )
    def _gather(src_h, idx_h, out_h, idx_v, rows_v, sem):
        wid = lax.axis_index("s") * NC + lax.axis_index("c")
        per = n_out // NW
        base = wid * per
        for c in range(per // _CH):
            pltpu.sync_copy(idx_h.at[pl.ds(base + c * _CH, _CH)], idx_v)
            pltpu.async_copy(src_h.at[idx_v], rows_v, sem).wait()
            pltpu.sync_copy(rows_v, out_h.at[pl.ds(base + c * _CH, _CH)])

    return _gather


def _ffn_body(be_ref, ba_ref, x_ref, s_ref, w1_ref, b1_ref, w2_ref, b2_ref,
              y_ref):
    g = pl.program_id(0)

    @pl.when(ba_ref[g] == 1)
    def _():
        h = jnp.dot(x_ref[...], w1_ref[0],
                    preferred_element_type=jnp.float32) + b1_ref[0, 0, :]
        y = jnp.dot(jax.nn.gelu(h), w2_ref[0],
                    preferred_element_type=jnp.float32)
        y_ref[...] = (y + b2_ref[0, 0, :]) * s_ref[...]


def _ffn(x_sorted, score_col, W1, b1, W2, b2, block_e, block_a):
    grid_spec = pltpu.PrefetchScalarGridSpec(
        num_scalar_prefetch=2,
        grid=(NB,),
        in_specs=[
            pl.BlockSpec((B, D), lambda g, be, ba: (g, 0)),
            pl.BlockSpec((B, 1), lambda g, be, ba: (g, 0)),
            pl.BlockSpec((1, D, F), lambda g, be, ba: (be[g], 0, 0)),
            pl.BlockSpec((1, 1, F), lambda g, be, ba: (be[g], 0, 0)),
            pl.BlockSpec((1, F, D), lambda g, be, ba: (be[g], 0, 0)),
            pl.BlockSpec((1, 1, D), lambda g, be, ba: (be[g], 0, 0)),
        ],
        out_specs=pl.BlockSpec((B, D), lambda g, be, ba: (g, 0)),
    )
    return pl.pallas_call(
        _ffn_body,
        grid_spec=grid_spec,
        out_shape=jax.ShapeDtypeStruct((PTOT, D), jnp.float32),
        compiler_params=pltpu.CompilerParams(
            dimension_semantics=("arbitrary",)),
    )(block_e, block_a, x_sorted, score_col, W1, b1.reshape(E, 1, F), W2,
      b2.reshape(E, 1, D))


_BT = 512


def _pair_add_body(g_ref, o_ref):
    o_ref[...] = g_ref[:, 0, :] + g_ref[:, 1, :]


def _pair_add(g):
    return pl.pallas_call(
        _pair_add_body,
        grid=(N // _BT,),
        in_specs=[pl.BlockSpec((_BT, K, D), lambda i: (i, 0, 0))],
        out_specs=pl.BlockSpec((_BT, D), lambda i: (i, 0)),
        out_shape=jax.ShapeDtypeStruct((N, D), jnp.float32),
    )(g)


def kernel(inp, w_gate, b_gate, W1, b1, W2, b2):
    gate_score, top_idx = _gate(inp, w_gate, b_gate)

    flat_idx = top_idx.reshape(-1)
    oh = (flat_idx[:, None] == jnp.arange(E)[None, :]).astype(jnp.int32)
    csum = jnp.cumsum(oh, axis=0)
    counts = csum[-1]
    rank = jnp.take_along_axis(csum, flat_idx[:, None], axis=1)[:, 0] - 1
    blocks_per_e = (counts + B - 1) // B
    padded_counts = blocks_per_e * B
    p_off = jnp.concatenate(
        [jnp.zeros((1,), jnp.int32),
         jnp.cumsum(padded_counts)[:-1].astype(jnp.int32)])
    pos_of_slot = p_off[flat_idx] + rank
    tok_of_pos = jnp.zeros((PTOT,), jnp.int32).at[pos_of_slot].set(
        jnp.arange(S, dtype=jnp.int32) // K)
    score_of_pos = jnp.zeros((PTOT,), jnp.float32).at[pos_of_slot].set(
        gate_score.reshape(-1))

    num_active = jnp.sum(blocks_per_e).astype(jnp.int32)
    be_raw = jnp.minimum(
        jnp.searchsorted(jnp.cumsum(blocks_per_e), jnp.arange(NB),
                         side="right"), E - 1).astype(jnp.int32)
    last_e = jnp.max(jnp.where(counts > 0, jnp.arange(E), 0)).astype(jnp.int32)
    block_e = jnp.where(jnp.arange(NB) < num_active, be_raw, last_e)
    block_a = (jnp.arange(NB) < num_active).astype(jnp.int32)

    x_sorted = _make_gather(PTOT)(inp, tok_of_pos)
    y_sorted = _ffn(x_sorted, score_of_pos[:, None], W1, b1, W2, b2,
                    block_e, block_a)
    g = _make_gather(S)(y_sorted, pos_of_slot)
    return _pair_add(g.reshape(N, K, D))


# speedup vs baseline: 2.4779x; 1.4665x over previous
import functools

import jax
import jax.numpy as jnp
from jax import lax
from jax.experimental import pallas as pl
from jax.experimental.pallas import tpu as pltpu
from jax.experimental.pallas import tpu_sc as plsc

E = 8
K = 2
D = 768
F = 3072
N = 2048
S = N * K
B = 256
NB = S // B + E
PTOT = NB * B
FB = 768
NF = F // FB

NC, NS = 2, 16
NW = NC * NS


def _gate_body(x_ref, wg_ref, bg_ref, gs_ref, ti_ref):
    logits = jnp.dot(x_ref[...], wg_ref[...],
                     preferred_element_type=jnp.float32) + bg_ref[...]
    col = lax.broadcasted_iota(jnp.int32, (N, E), 1)
    v0 = jnp.max(logits, axis=1, keepdims=True)
    i0 = jnp.min(jnp.where(logits == v0, col, E), axis=1, keepdims=True)
    masked = jnp.where(col == i0, -jnp.inf, logits)
    v1 = jnp.max(masked, axis=1, keepdims=True)
    i1 = jnp.min(jnp.where(masked == v1, col, E), axis=1, keepdims=True)
    e = jnp.exp(v1 - v0)
    s0 = 1.0 / (1.0 + e)
    gs_ref[...] = jnp.concatenate([s0, 1.0 - s0], axis=1)
    ti_ref[...] = jnp.concatenate([i0, i1], axis=1)


def _gate(inp, w_gate, b_gate):
    return pl.pallas_call(
        _gate_body,
        out_shape=(jax.ShapeDtypeStruct((N, K), jnp.float32),
                   jax.ShapeDtypeStruct((N, K), jnp.int32)),
    )(inp, w_gate, b_gate.reshape(1, E))


_CH = 64


_TPW = N // NW


@functools.cache
def _make_dispatch():
    @functools.partial(
        pl.kernel,
        out_type=jax.ShapeDtypeStruct((PTOT, D), jnp.float32),
        mesh=plsc.VectorSubcoreMesh(core_axis_name="c", subcore_axis_name="s",
                                    num_cores=NC, num_subcores=NS),
        scratch_types=[
            pltpu.VMEM((_TPW,), jnp.int32),
            pltpu.VMEM((_TPW,), jnp.int32),
            pltpu.VMEM((_TPW, D), jnp.float32),
            pltpu.SemaphoreType.DMA,
        ],
    )
    def _dispatch(inp_h, pe_h, po_h, x_h, pe_v, po_v, rows_v, sem):
        wid = lax.axis_index("s") * NC + lax.axis_index("c")
        base = wid * _TPW
        pltpu.sync_copy(inp_h.at[pl.ds(base, _TPW)], rows_v)
        pltpu.sync_copy(pe_h.at[pl.ds(base, _TPW)], pe_v)
        pltpu.sync_copy(po_h.at[pl.ds(base, _TPW)], po_v)
        a = pltpu.async_copy(rows_v, x_h.at[pe_v], sem)
        b = pltpu.async_copy(rows_v, x_h.at[po_v], sem)
        a.wait()
        b.wait()

    return _dispatch


@functools.cache
def _make_gather(n_out):
    @functools.partial(
        pl.kernel,
        out_type=jax.ShapeDtypeStruct((n_out, D), jnp.float32),
        mesh=plsc.VectorSubcoreMesh(core_axis_name="c", subcore_axis_name="s",
                                    num_cores=NC, num_subcores=NS),
        scratch_types=[
            pltpu.VMEM((_CH,), jnp.int32),
            pltpu.VMEM((_CH, D), jnp.float32),
            pltpu.SemaphoreType.DMA,
        ],
    )
    def _gather(src_h, idx_h, out_h, idx_v, rows_v, sem):
        wid = lax.axis_index("s") * NC + lax.axis_index("c")
        per = n_out // NW
        base = wid * per
        for c in range(per // _CH):
            pltpu.sync_copy(idx_h.at[pl.ds(base + c * _CH, _CH)], idx_v)
            pltpu.async_copy(src_h.at[idx_v], rows_v, sem).wait()
            pltpu.sync_copy(rows_v, out_h.at[pl.ds(base + c * _CH, _CH)])

    return _gather


def _ffn_body(be_ref, ba_ref, x_ref, s_ref, w1_ref, b1_ref, w2_ref, b2_ref,
              y_ref):
    g = pl.program_id(0)

    @pl.when(ba_ref[g] == 1)
    def _():
        h = jnp.dot(x_ref[...], w1_ref[0],
                    preferred_element_type=jnp.float32) + b1_ref[0, 0, :]
        y = jnp.dot(jax.nn.gelu(h), w2_ref[0],
                    preferred_element_type=jnp.float32)
        y_ref[...] = (y + b2_ref[0, 0, :]) * s_ref[...]


def _ffn(x_sorted, score_col, W1, b1, W2, b2, block_e, block_a):
    grid_spec = pltpu.PrefetchScalarGridSpec(
        num_scalar_prefetch=2,
        grid=(NB,),
        in_specs=[
            pl.BlockSpec((B, D), lambda g, be, ba: (g, 0)),
            pl.BlockSpec((B, 1), lambda g, be, ba: (g, 0)),
            pl.BlockSpec((1, D, F), lambda g, be, ba: (be[g], 0, 0)),
            pl.BlockSpec((1, 1, F), lambda g, be, ba: (be[g], 0, 0)),
            pl.BlockSpec((1, F, D), lambda g, be, ba: (be[g], 0, 0)),
            pl.BlockSpec((1, 1, D), lambda g, be, ba: (be[g], 0, 0)),
        ],
        out_specs=pl.BlockSpec((B, D), lambda g, be, ba: (g, 0)),
    )
    return pl.pallas_call(
        _ffn_body,
        grid_spec=grid_spec,
        out_shape=jax.ShapeDtypeStruct((PTOT, D), jnp.float32),
        compiler_params=pltpu.CompilerParams(
            dimension_semantics=("arbitrary",)),
    )(block_e, block_a, x_sorted, score_col, W1, b1.reshape(E, 1, F), W2,
      b2.reshape(E, 1, D))


_BT = 512


def _pair_add_body(g_ref, o_ref):
    o_ref[...] = g_ref[:, 0, :] + g_ref[:, 1, :]


def _pair_add(g):
    return pl.pallas_call(
        _pair_add_body,
        grid=(N // _BT,),
        in_specs=[pl.BlockSpec((_BT, K, D), lambda i: (i, 0, 0))],
        out_specs=pl.BlockSpec((_BT, D), lambda i: (i, 0)),
        out_shape=jax.ShapeDtypeStruct((N, D), jnp.float32),
    )(g)


def kernel(inp, w_gate, b_gate, W1, b1, W2, b2):
    gate_score, top_idx = _gate(inp, w_gate, b_gate)

    flat_idx = top_idx.reshape(-1)
    oh = (flat_idx[:, None] == jnp.arange(E)[None, :]).astype(jnp.int32)
    csum = jnp.cumsum(oh, axis=0)
    counts = csum[-1]
    rank = jnp.take_along_axis(csum, flat_idx[:, None], axis=1)[:, 0] - 1
    blocks_per_e = (counts + B - 1) // B
    padded_counts = blocks_per_e * B
    p_off = jnp.concatenate(
        [jnp.zeros((1,), jnp.int32),
         jnp.cumsum(padded_counts)[:-1].astype(jnp.int32)])
    pos_of_slot = p_off[flat_idx] + rank
    score_of_pos = jnp.zeros((PTOT,), jnp.float32).at[pos_of_slot].set(
        gate_score.reshape(-1))

    num_active = jnp.sum(blocks_per_e).astype(jnp.int32)
    be_raw = jnp.minimum(
        jnp.searchsorted(jnp.cumsum(blocks_per_e), jnp.arange(NB),
                         side="right"), E - 1).astype(jnp.int32)
    last_e = jnp.max(jnp.where(counts > 0, jnp.arange(E), 0)).astype(jnp.int32)
    block_e = jnp.where(jnp.arange(NB) < num_active, be_raw, last_e)
    block_a = (jnp.arange(NB) < num_active).astype(jnp.int32)

    x_sorted = _make_dispatch()(inp, pos_of_slot[0::2], pos_of_slot[1::2])
    y_sorted = _ffn(x_sorted, score_of_pos[:, None], W1, b1, W2, b2,
                    block_e, block_a)
    g = _make_gather(S)(y_sorted, pos_of_slot)
    return _pair_add(g.reshape(N, K, D))


# speedup vs baseline: 3.2312x; 1.3040x over previous
import functools

import jax
import jax.numpy as jnp
from jax import lax
from jax.experimental import pallas as pl
from jax.experimental.pallas import tpu as pltpu
from jax.experimental.pallas import tpu_sc as plsc

E = 8
K = 2
D = 768
F = 3072
N = 2048
S = N * K
B = 256
NB = S // B + E
PTOT = NB * B
FB = 768
NF = F // FB

NC, NS = 2, 16
NW = NC * NS


def _gate_body(x_ref, wg_ref, bg_ref, gs_ref, ti_ref):
    logits = jnp.dot(x_ref[...], wg_ref[...],
                     preferred_element_type=jnp.float32) + bg_ref[...]
    col = lax.broadcasted_iota(jnp.int32, (N, E), 1)
    v0 = jnp.max(logits, axis=1, keepdims=True)
    i0 = jnp.min(jnp.where(logits == v0, col, E), axis=1, keepdims=True)
    masked = jnp.where(col == i0, -jnp.inf, logits)
    v1 = jnp.max(masked, axis=1, keepdims=True)
    i1 = jnp.min(jnp.where(masked == v1, col, E), axis=1, keepdims=True)
    e = jnp.exp(v1 - v0)
    s0 = 1.0 / (1.0 + e)
    gs_ref[...] = jnp.concatenate([s0, 1.0 - s0], axis=1)
    ti_ref[...] = jnp.concatenate([i0, i1], axis=1)


def _gate(inp, w_gate, b_gate):
    return pl.pallas_call(
        _gate_body,
        out_shape=(jax.ShapeDtypeStruct((N, K), jnp.float32),
                   jax.ShapeDtypeStruct((N, K), jnp.int32)),
    )(inp, w_gate, b_gate.reshape(1, E))


_CH = 64


_TPW = N // NW


@functools.cache
def _make_dispatch():
    @functools.partial(
        pl.kernel,
        out_type=jax.ShapeDtypeStruct((PTOT, D), jnp.float32),
        mesh=plsc.VectorSubcoreMesh(core_axis_name="c", subcore_axis_name="s",
                                    num_cores=NC, num_subcores=NS),
        scratch_types=[
            pltpu.VMEM((_TPW,), jnp.int32),
            pltpu.VMEM((_TPW,), jnp.int32),
            pltpu.VMEM((_TPW, D), jnp.float32),
            pltpu.SemaphoreType.DMA,
        ],
    )
    def _dispatch(inp_h, pe_h, po_h, x_h, pe_v, po_v, rows_v, sem):
        wid = lax.axis_index("s") * NC + lax.axis_index("c")
        base = wid * _TPW
        pltpu.sync_copy(inp_h.at[pl.ds(base, _TPW)], rows_v)
        pltpu.sync_copy(pe_h.at[pl.ds(base, _TPW)], pe_v)
        pltpu.sync_copy(po_h.at[pl.ds(base, _TPW)], po_v)
        a = pltpu.async_copy(rows_v, x_h.at[pe_v], sem)
        b = pltpu.async_copy(rows_v, x_h.at[po_v], sem)
        a.wait()
        b.wait()

    return _dispatch


@functools.cache
def _make_combine_gather():
    @functools.partial(
        pl.kernel,
        out_type=jax.ShapeDtypeStruct((N, 2 * D), jnp.float32),
        mesh=plsc.VectorSubcoreMesh(core_axis_name="c", subcore_axis_name="s",
                                    num_cores=NC, num_subcores=NS),
        scratch_types=[
            pltpu.VMEM((_TPW,), jnp.int32),
            pltpu.VMEM((_TPW,), jnp.int32),
            pltpu.VMEM((_TPW, D), jnp.float32),
            pltpu.VMEM((_TPW, D), jnp.float32),
            pltpu.SemaphoreType.DMA,
        ],
    )
    def _cgather(y_h, pe_h, po_h, out_h, pe_v, po_v, a_v, b_v, sem):
        wid = lax.axis_index("s") * NC + lax.axis_index("c")
        tb = wid * _TPW
        pltpu.sync_copy(pe_h.at[pl.ds(tb, _TPW)], pe_v)
        pltpu.sync_copy(po_h.at[pl.ds(tb, _TPW)], po_v)
        a = pltpu.async_copy(y_h.at[pe_v], a_v, sem)
        b = pltpu.async_copy(y_h.at[po_v], b_v, sem)
        a.wait()
        b.wait()
        pltpu.sync_copy(a_v, out_h.at[pl.ds(tb, _TPW), pl.ds(0, D)])
        pltpu.sync_copy(b_v, out_h.at[pl.ds(tb, _TPW), pl.ds(D, D)])

    return _cgather


def _ffn_body(be_ref, ba_ref, x_ref, w1_ref, b1_ref, w2_ref, b2_ref, y_ref):
    g = pl.program_id(0)

    @pl.when(ba_ref[g] == 1)
    def _():
        h = jnp.dot(x_ref[...], w1_ref[0],
                    preferred_element_type=jnp.float32) + b1_ref[0, 0, :]
        y = jnp.dot(jax.nn.gelu(h), w2_ref[0],
                    preferred_element_type=jnp.float32)
        y_ref[...] = y + b2_ref[0, 0, :]


def _ffn(x_sorted, W1, b1, W2, b2, block_e, block_a):
    grid_spec = pltpu.PrefetchScalarGridSpec(
        num_scalar_prefetch=2,
        grid=(NB,),
        in_specs=[
            pl.BlockSpec((B, D), lambda g, be, ba: (g, 0)),
            pl.BlockSpec((1, D, F), lambda g, be, ba: (be[g], 0, 0)),
            pl.BlockSpec((1, 1, F), lambda g, be, ba: (be[g], 0, 0)),
            pl.BlockSpec((1, F, D), lambda g, be, ba: (be[g], 0, 0)),
            pl.BlockSpec((1, 1, D), lambda g, be, ba: (be[g], 0, 0)),
        ],
        out_specs=pl.BlockSpec((B, D), lambda g, be, ba: (g, 0)),
    )
    return pl.pallas_call(
        _ffn_body,
        grid_spec=grid_spec,
        out_shape=jax.ShapeDtypeStruct((PTOT, D), jnp.float32),
        compiler_params=pltpu.CompilerParams(
            dimension_semantics=("arbitrary",)),
    )(block_e, block_a, x_sorted, W1, b1.reshape(E, 1, F), W2,
      b2.reshape(E, 1, D))


_BT = 512


def _pair_add_body(g_ref, s_ref, o_ref):
    o_ref[...] = (g_ref[:, :D] * s_ref[:, 0:1] +
                  g_ref[:, D:] * s_ref[:, 1:2])


def _pair_add(g, gate_score):
    return pl.pallas_call(
        _pair_add_body,
        grid=(N // _BT,),
        in_specs=[pl.BlockSpec((_BT, 2 * D), lambda i: (i, 0)),
                  pl.BlockSpec((_BT, K), lambda i: (i, 0))],
        out_specs=pl.BlockSpec((_BT, D), lambda i: (i, 0)),
        out_shape=jax.ShapeDtypeStruct((N, D), jnp.float32),
    )(g, gate_score)


def kernel(inp, w_gate, b_gate, W1, b1, W2, b2):
    gate_score, top_idx = _gate(inp, w_gate, b_gate)

    flat_idx = top_idx.reshape(-1)
    oh = (flat_idx[:, None] == jnp.arange(E)[None, :]).astype(jnp.int32)
    csum = jnp.cumsum(oh, axis=0)
    counts = csum[-1]
    rank = jnp.take_along_axis(csum, flat_idx[:, None], axis=1)[:, 0] - 1
    blocks_per_e = (counts + B - 1) // B
    padded_counts = blocks_per_e * B
    p_off = jnp.concatenate(
        [jnp.zeros((1,), jnp.int32),
         jnp.cumsum(padded_counts)[:-1].astype(jnp.int32)])
    pos_of_slot = p_off[flat_idx] + rank

    num_active = jnp.sum(blocks_per_e).astype(jnp.int32)
    be_raw = jnp.minimum(
        jnp.searchsorted(jnp.cumsum(blocks_per_e), jnp.arange(NB),
                         side="right"), E - 1).astype(jnp.int32)
    last_e = jnp.max(jnp.where(counts > 0, jnp.arange(E), 0)).astype(jnp.int32)
    block_e = jnp.where(jnp.arange(NB) < num_active, be_raw, last_e)
    block_a = (jnp.arange(NB) < num_active).astype(jnp.int32)

    pe, po = pos_of_slot[0::2], pos_of_slot[1::2]
    x_sorted = _make_dispatch()(inp, pe, po)
    y_sorted = _ffn(x_sorted, W1, b1, W2, b2, block_e, block_a)
    g = _make_combine_gather()(y_sorted, pe, po)
    return _pair_add(g, gate_score)


# speedup vs baseline: 3.4508x; 1.0679x over previous
import functools

import jax
import jax.numpy as jnp
from jax import lax
from jax.experimental import pallas as pl
from jax.experimental.pallas import tpu as pltpu
from jax.experimental.pallas import tpu_sc as plsc

E = 8
K = 2
D = 768
F = 3072
N = 2048
S = N * K
B = 256
NB = S // B + E
PTOT = NB * B
FB = 768
NF = F // FB

NC, NS = 2, 16
NW = NC * NS


def _cumsum_rows(x, n):
    c = x
    sh = 1
    while sh < n:
        c = c + jnp.concatenate(
            [jnp.zeros((sh, x.shape[1]), x.dtype), c[:-sh]], axis=0)
        sh *= 2
    return c


def _cumsum_lanes(x, n):
    c = x
    sh = 1
    while sh < n:
        c = c + jnp.concatenate(
            [jnp.zeros((x.shape[0], sh), x.dtype), c[:, :-sh]], axis=1)
        sh *= 2
    return c


def _gate_body(x_ref, wg_ref, bg_ref, gs_ref, pp_ref, be_ref, ba_ref):
    logits = jnp.dot(x_ref[...], wg_ref[...],
                     preferred_element_type=jnp.float32) + bg_ref[...]
    col = lax.broadcasted_iota(jnp.int32, (N, E), 1)
    v0 = jnp.max(logits, axis=1, keepdims=True)
    i0 = jnp.min(jnp.where(logits == v0, col, E), axis=1, keepdims=True)
    masked = jnp.where(col == i0, -jnp.inf, logits)
    v1 = jnp.max(masked, axis=1, keepdims=True)
    i1 = jnp.min(jnp.where(masked == v1, col, E), axis=1, keepdims=True)
    e = jnp.exp(v1 - v0)
    s0 = 1.0 / (1.0 + e)
    gs_ref[...] = jnp.concatenate([s0, 1.0 - s0], axis=1)

    oh0 = (col == i0).astype(jnp.int32)
    oh1 = (col == i1).astype(jnp.int32)
    both = oh0 + oh1
    tot = jnp.sum(both, axis=0, keepdims=True)
    cex = _cumsum_rows(both, N) - both
    blocks_per_e = (tot + B - 1) // B
    cumb = _cumsum_lanes(blocks_per_e, E)
    p_off = jnp.concatenate(
        [jnp.zeros((1, 1), jnp.int32), cumb[:, :-1]], axis=1) * B
    rank0 = jnp.sum(jnp.where(col == i0, cex, 0), axis=1, keepdims=True)
    rank1 = jnp.sum(jnp.where(col == i1, cex, 0), axis=1, keepdims=True)
    off0 = jnp.sum(jnp.where(col == i0, p_off, 0), axis=1, keepdims=True)
    off1 = jnp.sum(jnp.where(col == i1, p_off, 0), axis=1, keepdims=True)
    pp_ref[...] = jnp.concatenate([off0 + rank0, off1 + rank1], axis=1)

    num_active = cumb[0, E - 1]
    gcol = lax.broadcasted_iota(jnp.int32, (NB, E), 0)
    be_raw = jnp.sum((gcol >= jnp.broadcast_to(cumb, (NB, E))).astype(
        jnp.int32), axis=1, keepdims=True)
    ecol = lax.broadcasted_iota(jnp.int32, (NB, E), 1)
    last_e = jnp.max(jnp.where(jnp.broadcast_to(tot, (NB, E)) > 0, ecol, 0),
                     axis=1, keepdims=True)
    gid = lax.broadcasted_iota(jnp.int32, (NB, 1), 0)
    be_ref[...] = jnp.where(gid < num_active,
                            jnp.minimum(be_raw, E - 1), last_e)
    ba_ref[...] = (gid < num_active).astype(jnp.int32)


def _gate(inp, w_gate, b_gate):
    return pl.pallas_call(
        _gate_body,
        out_shape=(jax.ShapeDtypeStruct((N, K), jnp.float32),
                   jax.ShapeDtypeStruct((N, K), jnp.int32),
                   jax.ShapeDtypeStruct((NB, 1), jnp.int32),
                   jax.ShapeDtypeStruct((NB, 1), jnp.int32)),
    )(inp, w_gate, b_gate.reshape(1, E))


_CH = 64


_TPW = N // NW


@functools.cache
def _make_dispatch():
    @functools.partial(
        pl.kernel,
        out_type=jax.ShapeDtypeStruct((PTOT, D), jnp.float32),
        mesh=plsc.VectorSubcoreMesh(core_axis_name="c", subcore_axis_name="s",
                                    num_cores=NC, num_subcores=NS),
        scratch_types=[
            pltpu.VMEM((_TPW,), jnp.int32),
            pltpu.VMEM((_TPW,), jnp.int32),
            pltpu.VMEM((_TPW, D), jnp.float32),
            pltpu.SemaphoreType.DMA,
        ],
    )
    def _dispatch(inp_h, pe_h, po_h, x_h, pe_v, po_v, rows_v, sem):
        wid = lax.axis_index("s") * NC + lax.axis_index("c")
        base = wid * _TPW
        pltpu.sync_copy(inp_h.at[pl.ds(base, _TPW)], rows_v)
        pltpu.sync_copy(pe_h.at[pl.ds(base, _TPW)], pe_v)
        pltpu.sync_copy(po_h.at[pl.ds(base, _TPW)], po_v)
        a = pltpu.async_copy(rows_v, x_h.at[pe_v], sem)
        b = pltpu.async_copy(rows_v, x_h.at[po_v], sem)
        a.wait()
        b.wait()

    return _dispatch


@functools.cache
def _make_combine_gather():
    @functools.partial(
        pl.kernel,
        out_type=jax.ShapeDtypeStruct((N, 2 * D), jnp.float32),
        mesh=plsc.VectorSubcoreMesh(core_axis_name="c", subcore_axis_name="s",
                                    num_cores=NC, num_subcores=NS),
        scratch_types=[
            pltpu.VMEM((_TPW,), jnp.int32),
            pltpu.VMEM((_TPW,), jnp.int32),
            pltpu.VMEM((_TPW, D), jnp.float32),
            pltpu.VMEM((_TPW, D), jnp.float32),
            pltpu.SemaphoreType.DMA,
        ],
    )
    def _cgather(y_h, pe_h, po_h, out_h, pe_v, po_v, a_v, b_v, sem):
        wid = lax.axis_index("s") * NC + lax.axis_index("c")
        tb = wid * _TPW
        pltpu.sync_copy(pe_h.at[pl.ds(tb, _TPW)], pe_v)
        pltpu.sync_copy(po_h.at[pl.ds(tb, _TPW)], po_v)
        a = pltpu.async_copy(y_h.at[pe_v], a_v, sem)
        b = pltpu.async_copy(y_h.at[po_v], b_v, sem)
        a.wait()
        b.wait()
        pltpu.sync_copy(a_v, out_h.at[pl.ds(tb, _TPW), pl.ds(0, D)])
        pltpu.sync_copy(b_v, out_h.at[pl.ds(tb, _TPW), pl.ds(D, D)])

    return _cgather


def _ffn_body(be_ref, ba_ref, x_ref, w1_ref, b1_ref, w2_ref, b2_ref, y_ref):
    g = pl.program_id(0)

    @pl.when(ba_ref[g] == 1)
    def _():
        h = jnp.dot(x_ref[...], w1_ref[0],
                    preferred_element_type=jnp.float32) + b1_ref[0, 0, :]
        y = jnp.dot(jax.nn.gelu(h), w2_ref[0],
                    preferred_element_type=jnp.float32)
        y_ref[...] = y + b2_ref[0, 0, :]


def _ffn(x_sorted, W1, b1, W2, b2, block_e, block_a):
    grid_spec = pltpu.PrefetchScalarGridSpec(
        num_scalar_prefetch=2,
        grid=(NB,),
        in_specs=[
            pl.BlockSpec((B, D), lambda g, be, ba: (g, 0)),
            pl.BlockSpec((1, D, F), lambda g, be, ba: (be[g], 0, 0)),
            pl.BlockSpec((1, 1, F), lambda g, be, ba: (be[g], 0, 0)),
            pl.BlockSpec((1, F, D), lambda g, be, ba: (be[g], 0, 0)),
            pl.BlockSpec((1, 1, D), lambda g, be, ba: (be[g], 0, 0)),
        ],
        out_specs=pl.BlockSpec((B, D), lambda g, be, ba: (g, 0)),
    )
    return pl.pallas_call(
        _ffn_body,
        grid_spec=grid_spec,
        out_shape=jax.ShapeDtypeStruct((PTOT, D), jnp.float32),
        compiler_params=pltpu.CompilerParams(
            dimension_semantics=("arbitrary",)),
    )(block_e, block_a, x_sorted, W1, b1.reshape(E, 1, F), W2,
      b2.reshape(E, 1, D))


_BT = 512


def _pair_add_body(g_ref, s_ref, o_ref):
    o_ref[...] = (g_ref[:, :D] * s_ref[:, 0:1] +
                  g_ref[:, D:] * s_ref[:, 1:2])


def _pair_add(g, gate_score):
    return pl.pallas_call(
        _pair_add_body,
        grid=(N // _BT,),
        in_specs=[pl.BlockSpec((_BT, 2 * D), lambda i: (i, 0)),
                  pl.BlockSpec((_BT, K), lambda i: (i, 0))],
        out_specs=pl.BlockSpec((_BT, D), lambda i: (i, 0)),
        out_shape=jax.ShapeDtypeStruct((N, D), jnp.float32),
    )(g, gate_score)


def kernel(inp, w_gate, b_gate, W1, b1, W2, b2):
    gate_score, pos_pair, be, ba = _gate(inp, w_gate, b_gate)
    pe, po = pos_pair[:, 0], pos_pair[:, 1]
    block_e, block_a = be[:, 0], ba[:, 0]
    x_sorted = _make_dispatch()(inp, pe, po)
    y_sorted = _ffn(x_sorted, W1, b1, W2, b2, block_e, block_a)
    g = _make_combine_gather()(y_sorted, pe, po)
    return _pair_add(g, gate_score)
